# Initial kernel scaffold; baseline (speedup 1.0000x reference)
import jax
import jax.numpy as jnp
from jax.experimental import pallas as pl


def kernel(users, items, user_memory_init, item_memory_init, W_ih, W_hh, b_ih, b_hh):
    raise NotImplementedError("write your pallas kernel here")


# speedup vs baseline: 14.1989x; 14.1989x over previous
import functools

import jax
import jax.numpy as jnp
from jax import lax
from jax.experimental import pallas as pl
from jax.experimental.pallas import tpu as pltpu
from jax.experimental.pallas import tpu_sc as plsc

E = 16
IDX_CHUNK = 128


def _tc_body(u0_ref, i0_ref, users_ref, items_ref, w_ref, b_ref, bhn_ref,
             out_u_ref, out_i_ref):
  S = users_ref.shape[0]
  W = w_ref[...]
  b = b_ref[...]
  bhn = bhn_ref[...]
  hist_u = []
  hist_i = []
  for t in range(S):
    ue = u0_ref[t]
    ie = i0_ref[t]
    ut = users_ref[t:t + 1, :]
    it = items_ref[t:t + 1, :]
    for tp in range(t):
      ue = jnp.where(users_ref[tp:tp + 1, :] == ut, hist_u[tp], ue)
      ie = jnp.where(items_ref[tp:tp + 1, :] == it, hist_i[tp], ie)
    xx = jnp.concatenate([ue, ie], axis=0)
    g = lax.dot_general(W, xx, (((1,), (0,)), ((), ())),
                        preferred_element_type=jnp.float32) + b
    rz = jax.nn.sigmoid(g[0:64, :])
    n_u = jnp.tanh(g[64:80, :] + rz[0:16, :] * bhn)
    n_i = jnp.tanh(g[80:96, :] + rz[32:48, :] * bhn)
    new_u = (1.0 - rz[16:32, :]) * n_u
    new_i = (1.0 - rz[48:64, :]) * n_i
    out_u_ref[t] = new_u
    out_i_ref[t] = new_i
    hist_u.append(new_u)
    hist_i.append(new_i)


def _make_sc_gather(total, num_rows_u, num_rows_i):
  info = plsc.get_sparse_core_info()
  nc, ns = info.num_cores, info.num_subcores
  nw = nc * ns
  per_w = total // nw
  n_chunks = per_w // IDX_CHUNK
  mesh = plsc.VectorSubcoreMesh(core_axis_name="c", subcore_axis_name="s")

  @functools.partial(
      pl.kernel, mesh=mesh,
      compiler_params=pltpu.CompilerParams(use_tc_tiling_on_sc=False),
      out_type=(jax.ShapeDtypeStruct((total, E), jnp.float32),
                jax.ShapeDtypeStruct((total, E), jnp.float32)),
      scratch_types=[
          pltpu.VMEM((n_chunks, IDX_CHUNK), jnp.int32),
          pltpu.VMEM((n_chunks, IDX_CHUNK), jnp.int32),
          pltpu.VMEM((per_w, E), jnp.float32),
          pltpu.VMEM((per_w, E), jnp.float32),
          pltpu.SemaphoreType.DMA,
          pltpu.SemaphoreType.DMA,
      ],
  )
  def gather_k(um_hbm, im_hbm, uidx_hbm, iidx_hbm, u0_hbm, i0_hbm,
               uidx_v, iidx_v, urows_v, irows_v, sem_u, sem_i):
    wid = lax.axis_index("s") * nc + lax.axis_index("c")
    base = wid * per_w
    pltpu.sync_copy(uidx_hbm.at[wid], uidx_v)
    pltpu.sync_copy(iidx_hbm.at[wid], iidx_v)
    copies = []
    for j in range(n_chunks):
      dst = pl.ds(j * IDX_CHUNK, IDX_CHUNK)
      copies.append(pltpu.async_copy(um_hbm.at[uidx_v.at[j]],
                                     urows_v.at[dst], sem_u))
      copies.append(pltpu.async_copy(im_hbm.at[iidx_v.at[j]],
                                     irows_v.at[dst], sem_i))
    for c in copies:
      c.wait()
    pltpu.sync_copy(urows_v, u0_hbm.at[pl.ds(base, per_w)])
    pltpu.sync_copy(irows_v, i0_hbm.at[pl.ds(base, per_w)])

  return gather_k, nw, n_chunks


def _pack_weights(W_ih, b_ih, b_hh):
  Wr, Wz, Wn = W_ih[0:E], W_ih[E:2 * E], W_ih[2 * E:3 * E]

  def swap(Wx):
    return jnp.concatenate([Wx[:, E:], Wx[:, :E]], axis=1)

  W3 = jnp.concatenate([Wr, Wz, swap(Wr), swap(Wz), Wn, swap(Wn)], axis=0)
  br = b_ih[0:E] + b_hh[0:E]
  bz = b_ih[E:2 * E] + b_hh[E:2 * E]
  bn = b_ih[2 * E:3 * E]
  b96 = jnp.concatenate([br, bz, br, bz, bn, bn])[:, None]
  bhn = b_hh[2 * E:3 * E][:, None]
  return W3, b96, bhn


def kernel(users, items, user_memory_init, item_memory_init,
           W_ih, W_hh, b_ih, b_hh):
  B, S = users.shape
  NU = user_memory_init.shape[1]
  NI = item_memory_init.shape[1]
  users = users.astype(jnp.int32)
  items = items.astype(jnp.int32)
  total = B * S

  gather_k, nw, n_chunks = _make_sc_gather(total, B * NU, B * NI)
  row_off = (jnp.arange(B, dtype=jnp.int32))[:, None]
  uidx = (users + row_off * NU).reshape(nw, n_chunks, IDX_CHUNK)
  iidx = (items + row_off * NI).reshape(nw, n_chunks, IDX_CHUNK)
  um_flat = user_memory_init.reshape(B * NU, E)
  im_flat = item_memory_init.reshape(B * NI, E)
  u0, i0 = gather_k(um_flat, im_flat, uidx, iidx)

  U0t = u0.reshape(B, S, E).transpose(1, 2, 0)
  I0t = i0.reshape(B, S, E).transpose(1, 2, 0)
  W3, b96, bhn = _pack_weights(W_ih, b_ih, b_hh)

  out_u_t, out_i_t = pl.pallas_call(
      _tc_body,
      out_shape=(jax.ShapeDtypeStruct((S, E, B), jnp.float32),
                 jax.ShapeDtypeStruct((S, E, B), jnp.float32)),
  )(U0t, I0t, users.T, items.T, W3, b96, bhn)

  return out_u_t.transpose(2, 0, 1), out_i_t.transpose(2, 0, 1)


# speedup vs baseline: 15.1386x; 1.0662x over previous
import functools

import jax
import jax.numpy as jnp
from jax import lax
from jax.experimental import pallas as pl
from jax.experimental.pallas import tpu as pltpu
from jax.experimental.pallas import tpu_sc as plsc

E = 16
IDX_CHUNK = 128


def _tc_body(u0_ref, i0_ref, users_ref, items_ref, w_ref, b_ref, bhn_ref,
             out_u_ref, out_i_ref):
  S = users_ref.shape[1]
  W = w_ref[...]
  b = b_ref[...]
  bhn = bhn_ref[...]
  ut_all = lax.transpose(users_ref[...], (1, 0))
  it_all = lax.transpose(items_ref[...], (1, 0))
  u0t = lax.transpose(u0_ref[...], (1, 0))
  i0t = lax.transpose(i0_ref[...], (1, 0))
  hist_u = []
  hist_i = []
  for t in range(S):
    ue = u0t[t * E:(t + 1) * E, :]
    ie = i0t[t * E:(t + 1) * E, :]
    ut = ut_all[t:t + 1, :]
    it = it_all[t:t + 1, :]
    for tp in range(t):
      ue = jnp.where(ut_all[tp:tp + 1, :] == ut, hist_u[tp], ue)
      ie = jnp.where(it_all[tp:tp + 1, :] == it, hist_i[tp], ie)
    xx = jnp.concatenate([ue, ie], axis=0)
    g = lax.dot_general(W, xx, (((1,), (0,)), ((), ())),
                        preferred_element_type=jnp.float32) + b
    rz = jax.nn.sigmoid(g[0:64, :])
    n_u = jnp.tanh(g[64:80, :] + rz[0:16, :] * bhn)
    n_i = jnp.tanh(g[80:96, :] + rz[32:48, :] * bhn)
    new_u = (1.0 - rz[16:32, :]) * n_u
    new_i = (1.0 - rz[48:64, :]) * n_i
    hist_u.append(new_u)
    hist_i.append(new_i)
  out_u_ref[...] = lax.transpose(jnp.concatenate(hist_u, axis=0), (1, 0))
  out_i_ref[...] = lax.transpose(jnp.concatenate(hist_i, axis=0), (1, 0))


def _make_sc_gather(total, num_rows_u, num_rows_i):
  info = plsc.get_sparse_core_info()
  nc, ns = info.num_cores, info.num_subcores
  nw = nc * ns
  per_w = total // nw
  n_chunks = per_w // IDX_CHUNK
  mesh = plsc.VectorSubcoreMesh(core_axis_name="c", subcore_axis_name="s")

  @functools.partial(
      pl.kernel, mesh=mesh,
      compiler_params=pltpu.CompilerParams(use_tc_tiling_on_sc=False),
      out_type=(jax.ShapeDtypeStruct((total, E), jnp.float32),
                jax.ShapeDtypeStruct((total, E), jnp.float32)),
      scratch_types=[
          pltpu.VMEM((n_chunks, IDX_CHUNK), jnp.int32),
          pltpu.VMEM((n_chunks, IDX_CHUNK), jnp.int32),
          pltpu.VMEM((per_w, E), jnp.float32),
          pltpu.VMEM((per_w, E), jnp.float32),
          pltpu.SemaphoreType.DMA,
          pltpu.SemaphoreType.DMA,
      ],
  )
  def gather_k(um_hbm, im_hbm, uidx_hbm, iidx_hbm, u0_hbm, i0_hbm,
               uidx_v, iidx_v, urows_v, irows_v, sem_u, sem_i):
    wid = lax.axis_index("s") * nc + lax.axis_index("c")
    base = wid * per_w
    pltpu.sync_copy(uidx_hbm.at[wid], uidx_v)
    pltpu.sync_copy(iidx_hbm.at[wid], iidx_v)
    copies = []
    for j in range(n_chunks):
      dst = pl.ds(j * IDX_CHUNK, IDX_CHUNK)
      copies.append(pltpu.async_copy(um_hbm.at[uidx_v.at[j]],
                                     urows_v.at[dst], sem_u))
      copies.append(pltpu.async_copy(im_hbm.at[iidx_v.at[j]],
                                     irows_v.at[dst], sem_i))
    for c in copies:
      c.wait()
    pltpu.sync_copy(urows_v, u0_hbm.at[pl.ds(base, per_w)])
    pltpu.sync_copy(irows_v, i0_hbm.at[pl.ds(base, per_w)])

  return gather_k, nw, n_chunks


def _pack_weights(W_ih, b_ih, b_hh):
  Wr, Wz, Wn = W_ih[0:E], W_ih[E:2 * E], W_ih[2 * E:3 * E]

  def swap(Wx):
    return jnp.concatenate([Wx[:, E:], Wx[:, :E]], axis=1)

  W3 = jnp.concatenate([Wr, Wz, swap(Wr), swap(Wz), Wn, swap(Wn)], axis=0)
  br = b_ih[0:E] + b_hh[0:E]
  bz = b_ih[E:2 * E] + b_hh[E:2 * E]
  bn = b_ih[2 * E:3 * E]
  b96 = jnp.concatenate([br, bz, br, bz, bn, bn])[:, None]
  bhn = b_hh[2 * E:3 * E][:, None]
  return W3, b96, bhn


def kernel(users, items, user_memory_init, item_memory_init,
           W_ih, W_hh, b_ih, b_hh):
  B, S = users.shape
  NU = user_memory_init.shape[1]
  NI = item_memory_init.shape[1]
  users = users.astype(jnp.int32)
  items = items.astype(jnp.int32)
  total = B * S

  gather_k, nw, n_chunks = _make_sc_gather(total, B * NU, B * NI)
  row_off = (jnp.arange(B, dtype=jnp.int32))[:, None]
  uidx = (users + row_off * NU).reshape(nw, n_chunks, IDX_CHUNK)
  iidx = (items + row_off * NI).reshape(nw, n_chunks, IDX_CHUNK)
  um_flat = user_memory_init.reshape(B * NU, E)
  im_flat = item_memory_init.reshape(B * NI, E)
  u0, i0 = gather_k(um_flat, im_flat, uidx, iidx)

  W3, b96, bhn = _pack_weights(W_ih, b_ih, b_hh)

  out_u, out_i = pl.pallas_call(
      _tc_body,
      out_shape=(jax.ShapeDtypeStruct((B, S * E), jnp.float32),
                 jax.ShapeDtypeStruct((B, S * E), jnp.float32)),
  )(u0.reshape(B, S * E), i0.reshape(B, S * E), users, items, W3, b96, bhn)

  return out_u.reshape(B, S, E), out_i.reshape(B, S, E)


# speedup vs baseline: 82.5657x; 5.4540x over previous
import functools

import jax
import jax.numpy as jnp
from jax import lax
from jax.experimental import pallas as pl
from jax.experimental.pallas import tpu as pltpu
from jax.experimental.pallas import tpu_sc as plsc

E = 16
CHUNK = 128
PACK = 128 // E


def _pack_body(m_ref, p_ref):
  ne, bblk = m_ref.shape
  rows = bblk * ne // 128
  p_ref[...] = lax.transpose(m_ref[...], (1, 0)).reshape(rows, 128)


def _pack_table(m_t2d, lanes=128):
  ne, B = m_t2d.shape
  grid = B // lanes
  rows_blk = lanes * ne // 128
  return pl.pallas_call(
      _pack_body,
      grid=(grid,),
      in_specs=[pl.BlockSpec((ne, lanes), lambda g: (0, g))],
      out_specs=pl.BlockSpec((rows_blk, 128), lambda g: (g, 0)),
      out_shape=jax.ShapeDtypeStruct((B * ne // 128, 128), jnp.float32),
      compiler_params=pltpu.CompilerParams(
          vmem_limit_bytes=100 * 1024 * 1024),
  )(m_t2d)


def _tc_body(u0_ref, i0_ref, users_ref, items_ref, w_ref, b_ref, bhn_ref,
             out_u_ref, out_i_ref):
  S = users_ref.shape[0]
  W = w_ref[...]
  b = b_ref[...]
  bhn = bhn_ref[...]
  ut_all = users_ref[...]
  it_all = items_ref[...]
  u0t = lax.transpose(u0_ref[...], (1, 0))
  i0t = lax.transpose(i0_ref[...], (1, 0))
  hist_u = []
  hist_i = []
  for t in range(S):
    ue = u0t[t * E:(t + 1) * E, :]
    ie = i0t[t * E:(t + 1) * E, :]
    ut = ut_all[t:t + 1, :]
    it = it_all[t:t + 1, :]
    for tp in range(t):
      ue = jnp.where(ut_all[tp:tp + 1, :] == ut, hist_u[tp], ue)
      ie = jnp.where(it_all[tp:tp + 1, :] == it, hist_i[tp], ie)
    xx = jnp.concatenate([ue, ie], axis=0)
    g = lax.dot_general(W, xx, (((1,), (0,)), ((), ())),
                        preferred_element_type=jnp.float32) + b
    rz = jax.nn.sigmoid(g[0:64, :])
    n_u = jnp.tanh(g[64:80, :] + rz[0:16, :] * bhn)
    n_i = jnp.tanh(g[80:96, :] + rz[32:48, :] * bhn)
    new_u = (1.0 - rz[16:32, :]) * n_u
    new_i = (1.0 - rz[48:64, :]) * n_i
    hist_u.append(new_u)
    hist_i.append(new_i)
  out_u_ref[...] = jnp.concatenate(hist_u, axis=0)
  out_i_ref[...] = jnp.concatenate(hist_i, axis=0)


def _make_sc_gather(total, rows_u, rows_i, variant="full"):
  info = plsc.get_sparse_core_info()
  nc, ns = info.num_cores, info.num_subcores
  nw = nc * ns
  per_w = total // nw
  n_chunks = per_w // CHUNK
  vper_chunk = CHUNK * E // 16
  orows_chunk = CHUNK // PACK
  mesh = plsc.VectorSubcoreMesh(core_axis_name="c", subcore_axis_name="s")

  @functools.partial(
      pl.kernel, mesh=mesh,
      compiler_params=pltpu.CompilerParams(use_tc_tiling_on_sc=True),
      out_type=(jax.ShapeDtypeStruct((total // PACK, 128), jnp.float32),
                jax.ShapeDtypeStruct((total // PACK, 128), jnp.float32)),
      scratch_types=[
          pltpu.VMEM((n_chunks, CHUNK), jnp.int32),
          pltpu.VMEM((n_chunks, CHUNK), jnp.int32),
          pltpu.VMEM((n_chunks, CHUNK), jnp.int32),
          pltpu.VMEM((n_chunks, CHUNK), jnp.int32),
          pltpu.VMEM((n_chunks, CHUNK), jnp.int32),
          pltpu.VMEM((n_chunks, CHUNK), jnp.int32),
          pltpu.VMEM((2, CHUNK, 128), jnp.float32),
          pltpu.VMEM((2, CHUNK, 128), jnp.float32),
          pltpu.VMEM((orows_chunk, 128), jnp.float32),
          pltpu.VMEM((orows_chunk, 128), jnp.float32),
          pltpu.SemaphoreType.DMA,
          pltpu.SemaphoreType.DMA,
      ],
  )
  def gather_k(um_hbm, im_hbm, uidx_hbm, iidx_hbm, u0_hbm, i0_hbm,
               uf_v, if_v, ug_v, ig_v, us_v, is_v,
               ugrp_v, igrp_v, uout_v, iout_v, sem_u, sem_i):
    wid = lax.axis_index("s") * nc + lax.axis_index("c")
    pltpu.sync_copy(uidx_hbm.at[wid], uf_v)
    pltpu.sync_copy(iidx_hbm.at[wid], if_v)
    for j in range(n_chunks):
      for k in range(CHUNK // 16):
        sl = pl.ds(k * 16, 16)
        fu = uf_v[j, sl]
        fi = if_v[j, sl]
        ug_v[j, sl] = jnp.right_shift(fu, 3)
        ig_v[j, sl] = jnp.right_shift(fi, 3)
        us_v[j, sl] = jnp.left_shift(jnp.bitwise_and(fu, 7), 4)
        is_v[j, sl] = jnp.left_shift(jnp.bitwise_and(fi, 7), 4)

    def extract(grp, offs_v, out, j):
      if variant == "noext":
        return
      for g in range(CHUNK // 16):
        ovec = offs_v[j, pl.ds(g * 16, 16)]
        for k in range(16):
          i = g * 16 + k
          vals = grp[i, pl.ds(ovec[k], 16)]
          out[i // 8, pl.ds((i % 8) * 16, 16)] = vals

    def fire(j):
      cu = pltpu.async_copy(um_hbm.at[ug_v.at[j]], ugrp_v.at[j % 2], sem_u)
      ci = pltpu.async_copy(im_hbm.at[ig_v.at[j]], igrp_v.at[j % 2], sem_i)
      return cu, ci

    def drain_extract_store(j, cu, ci):
      cu.wait()
      extract(ugrp_v.at[j % 2], us_v, uout_v, j)
      pltpu.sync_copy(
          uout_v, u0_hbm.at[pl.ds(wid * (per_w // PACK) + j * orows_chunk,
                                  orows_chunk)])
      ci.wait()
      extract(igrp_v.at[j % 2], is_v, iout_v, j)
      pltpu.sync_copy(
          iout_v, i0_hbm.at[pl.ds(wid * (per_w // PACK) + j * orows_chunk,
                                  orows_chunk)])

    pend = fire(0)
    for j in range(n_chunks):
      nxt = fire(j + 1) if j + 1 < n_chunks else None
      drain_extract_store(j, *pend)
      pend = nxt

  return gather_k, nw, n_chunks


def _pack_weights(W_ih, b_ih, b_hh):
  Wr, Wz, Wn = W_ih[0:E], W_ih[E:2 * E], W_ih[2 * E:3 * E]

  def swap(Wx):
    return jnp.concatenate([Wx[:, E:], Wx[:, :E]], axis=1)

  W3 = jnp.concatenate([Wr, Wz, swap(Wr), swap(Wz), Wn, swap(Wn)], axis=0)
  br = b_ih[0:E] + b_hh[0:E]
  bz = b_ih[E:2 * E] + b_hh[E:2 * E]
  bn = b_ih[2 * E:3 * E]
  b96 = jnp.concatenate([br, bz, br, bz, bn, bn])[:, None]
  bhn = b_hh[2 * E:3 * E][:, None]
  return W3, b96, bhn


def kernel(users, items, user_memory_init, item_memory_init,
           W_ih, W_hh, b_ih, b_hh):
  B, S = users.shape
  NU = user_memory_init.shape[1]
  NI = item_memory_init.shape[1]
  users = users.astype(jnp.int32)
  items = items.astype(jnp.int32)
  total = B * S

  um_t2d = user_memory_init.transpose(1, 2, 0).reshape(NU * E, B)
  im_t2d = item_memory_init.transpose(1, 2, 0).reshape(NI * E, B)
  um_p = _pack_table(um_t2d)
  im_p = _pack_table(im_t2d)

  gather_k, nw, n_chunks = _make_sc_gather(total, B * NU, B * NI)
  row_off = (jnp.arange(B, dtype=jnp.int32))[:, None]
  uidx = (users + row_off * NU).reshape(nw, n_chunks, CHUNK)
  iidx = (items + row_off * NI).reshape(nw, n_chunks, CHUNK)
  u0p, i0p = gather_k(um_p, im_p, uidx, iidx)

  W3, b96, bhn = _pack_weights(W_ih, b_ih, b_hh)

  out_u_t, out_i_t = pl.pallas_call(
      _tc_body,
      out_shape=(jax.ShapeDtypeStruct((S * E, B), jnp.float32),
                 jax.ShapeDtypeStruct((S * E, B), jnp.float32)),
  )(u0p.reshape(B, S * E), i0p.reshape(B, S * E), users.T, items.T,
    W3, b96, bhn)

  out_u = out_u_t.reshape(S, E, B).transpose(2, 0, 1)
  out_i = out_i_t.reshape(S, E, B).transpose(2, 0, 1)
  return out_u, out_i


# speedup vs baseline: 82.6820x; 1.0014x over previous
import functools

import jax
import jax.numpy as jnp
from jax import lax
from jax.experimental import pallas as pl
from jax.experimental.pallas import tpu as pltpu
from jax.experimental.pallas import tpu_sc as plsc

E = 16
CHUNK = 128
PACK = 128 // E


def _pack_body(m_ref, p_ref):
  ne, bblk = m_ref.shape
  rows = bblk * ne // 128
  p_ref[...] = lax.transpose(m_ref[...], (1, 0)).reshape(rows, 128)


def _pack_table(m_t2d, lanes=128):
  ne, B = m_t2d.shape
  grid = B // lanes
  rows_blk = lanes * ne // 128
  return pl.pallas_call(
      _pack_body,
      grid=(grid,),
      in_specs=[pl.BlockSpec((ne, lanes), lambda g: (0, g))],
      out_specs=pl.BlockSpec((rows_blk, 128), lambda g: (g, 0)),
      out_shape=jax.ShapeDtypeStruct((B * ne // 128, 128), jnp.float32),
      compiler_params=pltpu.CompilerParams(
          vmem_limit_bytes=100 * 1024 * 1024),
  )(m_t2d)


def _tc_body(u0_ref, i0_ref, users_ref, items_ref, w_ref, b_ref, bhn_ref,
             out_u_ref, out_i_ref):
  S = users_ref.shape[0]
  W = w_ref[...]
  b = b_ref[...]
  bhn = bhn_ref[...]
  ut_all = users_ref[...]
  it_all = items_ref[...]
  u0t = lax.transpose(u0_ref[...], (1, 0))
  i0t = lax.transpose(i0_ref[...], (1, 0))
  hist_u = []
  hist_i = []
  for t in range(S):
    ue = u0t[t * E:(t + 1) * E, :]
    ie = i0t[t * E:(t + 1) * E, :]
    ut = ut_all[t:t + 1, :]
    it = it_all[t:t + 1, :]
    for tp in range(t):
      ue = jnp.where(ut_all[tp:tp + 1, :] == ut, hist_u[tp], ue)
      ie = jnp.where(it_all[tp:tp + 1, :] == it, hist_i[tp], ie)
    xx = jnp.concatenate([ue, ie], axis=0)
    g = lax.dot_general(W, xx, (((1,), (0,)), ((), ())),
                        preferred_element_type=jnp.float32) + b
    rz = jax.nn.sigmoid(g[0:64, :])
    n_u = jnp.tanh(g[64:80, :] + rz[0:16, :] * bhn)
    n_i = jnp.tanh(g[80:96, :] + rz[32:48, :] * bhn)
    new_u = (1.0 - rz[16:32, :]) * n_u
    new_i = (1.0 - rz[48:64, :]) * n_i
    hist_u.append(new_u)
    hist_i.append(new_i)
  out_u_ref[...] = jnp.concatenate(hist_u, axis=0)
  out_i_ref[...] = jnp.concatenate(hist_i, axis=0)


def _make_sc_gather(total, rows_u, rows_i):
  info = plsc.get_sparse_core_info()
  nc, ns = info.num_cores, info.num_subcores
  nw = nc * ns
  per_w = total // nw
  n_chunks = per_w // CHUNK
  vper_chunk = CHUNK * E // 16
  orows_chunk = CHUNK // PACK
  mesh = plsc.VectorSubcoreMesh(core_axis_name="c", subcore_axis_name="s")

  @functools.partial(
      pl.kernel, mesh=mesh,
      compiler_params=pltpu.CompilerParams(use_tc_tiling_on_sc=True),
      out_type=(jax.ShapeDtypeStruct((total // PACK, 128), jnp.float32),
                jax.ShapeDtypeStruct((total // PACK, 128), jnp.float32)),
      scratch_types=[
          pltpu.VMEM((n_chunks, CHUNK), jnp.int32),
          pltpu.VMEM((n_chunks, CHUNK), jnp.int32),
          pltpu.VMEM((n_chunks, CHUNK), jnp.int32),
          pltpu.VMEM((n_chunks, CHUNK), jnp.int32),
          pltpu.VMEM((n_chunks, CHUNK), jnp.int32),
          pltpu.VMEM((n_chunks, CHUNK), jnp.int32),
          pltpu.VMEM((2, CHUNK, 128), jnp.float32),
          pltpu.VMEM((2, CHUNK, 128), jnp.float32),
          pltpu.VMEM((orows_chunk, 128), jnp.float32),
          pltpu.VMEM((orows_chunk, 128), jnp.float32),
          pltpu.SemaphoreType.DMA,
          pltpu.SemaphoreType.DMA,
      ],
  )
  def gather_k(um_hbm, im_hbm, uidx_hbm, iidx_hbm, u0_hbm, i0_hbm,
               uf_v, if_v, ug_v, ig_v, us_v, is_v,
               ugrp_v, igrp_v, uout_v, iout_v, sem_u, sem_i):
    wid = lax.axis_index("s") * nc + lax.axis_index("c")
    pltpu.sync_copy(uidx_hbm.at[wid], uf_v)
    pltpu.sync_copy(iidx_hbm.at[wid], if_v)
    for j in range(n_chunks):
      for k in range(CHUNK // 16):
        sl = pl.ds(k * 16, 16)
        fu = uf_v[j, sl]
        fi = if_v[j, sl]
        ug_v[j, sl] = jnp.right_shift(fu, 3)
        ig_v[j, sl] = jnp.right_shift(fi, 3)
        us_v[j, sl] = jnp.left_shift(jnp.bitwise_and(fu, 7), 4)
        is_v[j, sl] = jnp.left_shift(jnp.bitwise_and(fi, 7), 4)

    def extract(grp, offs_v, out, j):
      for g in range(CHUNK // 16):
        ovec = offs_v[j, pl.ds(g * 16, 16)]
        for k in range(16):
          i = g * 16 + k
          vals = grp[i, pl.ds(ovec[k], 16)]
          out[i // 8, pl.ds((i % 8) * 16, 16)] = vals

    def fire(j):
      cu = pltpu.async_copy(um_hbm.at[ug_v.at[j]], ugrp_v.at[j % 2], sem_u)
      ci = pltpu.async_copy(im_hbm.at[ig_v.at[j]], igrp_v.at[j % 2], sem_i)
      return cu, ci

    def drain_extract_store(j, cu, ci):
      cu.wait()
      extract(ugrp_v.at[j % 2], us_v, uout_v, j)
      pltpu.sync_copy(
          uout_v, u0_hbm.at[pl.ds(wid * (per_w // PACK) + j * orows_chunk,
                                  orows_chunk)])
      ci.wait()
      extract(igrp_v.at[j % 2], is_v, iout_v, j)
      pltpu.sync_copy(
          iout_v, i0_hbm.at[pl.ds(wid * (per_w // PACK) + j * orows_chunk,
                                  orows_chunk)])

    pend = fire(0)
    for j in range(n_chunks):
      nxt = fire(j + 1) if j + 1 < n_chunks else None
      drain_extract_store(j, *pend)
      pend = nxt

  return gather_k, nw, n_chunks


def _pack_weights(W_ih, b_ih, b_hh):
  Wr, Wz, Wn = W_ih[0:E], W_ih[E:2 * E], W_ih[2 * E:3 * E]

  def swap(Wx):
    return jnp.concatenate([Wx[:, E:], Wx[:, :E]], axis=1)

  W3 = jnp.concatenate([Wr, Wz, swap(Wr), swap(Wz), Wn, swap(Wn)], axis=0)
  br = b_ih[0:E] + b_hh[0:E]
  bz = b_ih[E:2 * E] + b_hh[E:2 * E]
  bn = b_ih[2 * E:3 * E]
  b96 = jnp.concatenate([br, bz, br, bz, bn, bn])[:, None]
  bhn = b_hh[2 * E:3 * E][:, None]
  return W3, b96, bhn


def kernel(users, items, user_memory_init, item_memory_init,
           W_ih, W_hh, b_ih, b_hh):
  B, S = users.shape
  NU = user_memory_init.shape[1]
  NI = item_memory_init.shape[1]
  users = users.astype(jnp.int32)
  items = items.astype(jnp.int32)
  total = B * S

  um_t2d = user_memory_init.transpose(1, 2, 0).reshape(NU * E, B)
  im_t2d = item_memory_init.transpose(1, 2, 0).reshape(NI * E, B)
  um_p = _pack_table(um_t2d)
  im_p = _pack_table(im_t2d)

  gather_k, nw, n_chunks = _make_sc_gather(total, B * NU, B * NI)
  row_off = (jnp.arange(B, dtype=jnp.int32))[:, None]
  uidx = (users + row_off * NU).reshape(nw, n_chunks, CHUNK)
  iidx = (items + row_off * NI).reshape(nw, n_chunks, CHUNK)
  u0p, i0p = gather_k(um_p, im_p, uidx, iidx)

  W3, b96, bhn = _pack_weights(W_ih, b_ih, b_hh)

  out_u_t, out_i_t = pl.pallas_call(
      _tc_body,
      out_shape=(jax.ShapeDtypeStruct((S * E, B), jnp.float32),
                 jax.ShapeDtypeStruct((S * E, B), jnp.float32)),
  )(u0p.reshape(B, S * E), i0p.reshape(B, S * E), users.T, items.T,
    W3, b96, bhn)

  out_u = out_u_t.reshape(S, E, B).transpose(2, 0, 1)
  out_i = out_i_t.reshape(S, E, B).transpose(2, 0, 1)
  return out_u, out_i


# speedup vs baseline: 83.6606x; 1.0118x over previous
import functools

import jax
import jax.numpy as jnp
from jax import lax
from jax.experimental import pallas as pl
from jax.experimental.pallas import tpu as pltpu
from jax.experimental.pallas import tpu_sc as plsc

E = 16
CHUNK = 128
PACK = 128 // E


def _pack_body(m_ref, p_ref):
  ne, bblk = m_ref.shape
  rows = bblk * ne // 128
  p_ref[...] = lax.transpose(m_ref[...], (1, 0)).reshape(rows, 128)


def _pack_table(m_t2d, lanes=128):
  ne, B = m_t2d.shape
  grid = B // lanes
  rows_blk = lanes * ne // 128
  return pl.pallas_call(
      _pack_body,
      grid=(grid,),
      in_specs=[pl.BlockSpec((ne, lanes), lambda g: (0, g))],
      out_specs=pl.BlockSpec((rows_blk, 128), lambda g: (g, 0)),
      out_shape=jax.ShapeDtypeStruct((B * ne // 128, 128), jnp.float32),
      compiler_params=pltpu.CompilerParams(
          vmem_limit_bytes=100 * 1024 * 1024),
  )(m_t2d)


def _tc_body(u0_ref, i0_ref, users_ref, items_ref, w_ref, b_ref, bhn_ref,
             out_u_ref, out_i_ref):
  S = users_ref.shape[0]
  W = w_ref[...]
  b = b_ref[...]
  bhn = bhn_ref[...]
  ut_all = users_ref[...]
  it_all = items_ref[...]
  u0t = lax.transpose(u0_ref[...], (1, 0))
  i0t = lax.transpose(i0_ref[...], (1, 0))
  hist_u = []
  hist_i = []
  for t in range(S):
    ue = u0t[t * E:(t + 1) * E, :]
    ie = i0t[t * E:(t + 1) * E, :]
    ut = ut_all[t:t + 1, :]
    it = it_all[t:t + 1, :]
    for tp in range(t):
      ue = jnp.where(ut_all[tp:tp + 1, :] == ut, hist_u[tp], ue)
      ie = jnp.where(it_all[tp:tp + 1, :] == it, hist_i[tp], ie)
    xx = jnp.concatenate([ue, ie], axis=0)
    g = lax.dot_general(W, xx, (((1,), (0,)), ((), ())),
                        preferred_element_type=jnp.float32) + b
    rz = jax.nn.sigmoid(g[0:64, :])
    n_u = jnp.tanh(g[64:80, :] + rz[0:16, :] * bhn)
    n_i = jnp.tanh(g[80:96, :] + rz[32:48, :] * bhn)
    new_u = (1.0 - rz[16:32, :]) * n_u
    new_i = (1.0 - rz[48:64, :]) * n_i
    hist_u.append(new_u)
    hist_i.append(new_i)
  out_u_ref[...] = jnp.concatenate(hist_u, axis=0)
  out_i_ref[...] = jnp.concatenate(hist_i, axis=0)


def _make_sc_gather(total, rows_u, rows_i):
  info = plsc.get_sparse_core_info()
  nc, ns = info.num_cores, info.num_subcores
  nw = nc * ns
  per_w = total // nw
  n_chunks = per_w // CHUNK
  vper_chunk = CHUNK * E // 16
  orows_chunk = CHUNK // PACK
  mesh = plsc.VectorSubcoreMesh(core_axis_name="c", subcore_axis_name="s")

  @functools.partial(
      pl.kernel, mesh=mesh,
      compiler_params=pltpu.CompilerParams(use_tc_tiling_on_sc=True),
      out_type=jax.ShapeDtypeStruct((total // PACK, 128), jnp.float32),
      scratch_types=[
          pltpu.VMEM((n_chunks, CHUNK), jnp.int32),
          pltpu.VMEM((n_chunks, CHUNK), jnp.int32),
          pltpu.VMEM((n_chunks, CHUNK), jnp.int32),
          pltpu.VMEM((2, CHUNK, 128), jnp.float32),
          pltpu.VMEM((orows_chunk, 128), jnp.float32),
          pltpu.SemaphoreType.DMA,
      ],
  )
  def gather_k(um_hbm, uidx_hbm, u0_hbm,
               uf_v, ug_v, us_v, ugrp_v, uout_v, sem_u):
    wid = lax.axis_index("s") * nc + lax.axis_index("c")
    pltpu.sync_copy(uidx_hbm.at[wid], uf_v)
    for j in range(n_chunks):
      for k in range(CHUNK // 16):
        sl = pl.ds(k * 16, 16)
        fu = uf_v[j, sl]
        ug_v[j, sl] = jnp.right_shift(fu, 3)
        us_v[j, sl] = jnp.left_shift(jnp.bitwise_and(fu, 7), 4)

    def extract(grp, offs_v, out, j):
      for g in range(CHUNK // 16):
        ovec = offs_v[j, pl.ds(g * 16, 16)]
        for k in range(16):
          i = g * 16 + k
          vals = grp[i, pl.ds(ovec[k], 16)]
          out[i // 8, pl.ds((i % 8) * 16, 16)] = vals

    def fire(j):
      return pltpu.async_copy(um_hbm.at[ug_v.at[j]], ugrp_v.at[j % 2], sem_u)

    def drain_extract_store(j, cu):
      cu.wait()
      extract(ugrp_v.at[j % 2], us_v, uout_v, j)
      pltpu.sync_copy(
          uout_v, u0_hbm.at[pl.ds(wid * (per_w // PACK) + j * orows_chunk,
                                  orows_chunk)])

    pend = fire(0)
    for j in range(n_chunks):
      nxt = fire(j + 1) if j + 1 < n_chunks else None
      drain_extract_store(j, pend)
      pend = nxt

  return gather_k, nw, n_chunks


def _pack_weights(W_ih, b_ih, b_hh):
  Wr, Wz, Wn = W_ih[0:E], W_ih[E:2 * E], W_ih[2 * E:3 * E]

  def swap(Wx):
    return jnp.concatenate([Wx[:, E:], Wx[:, :E]], axis=1)

  W3 = jnp.concatenate([Wr, Wz, swap(Wr), swap(Wz), Wn, swap(Wn)], axis=0)
  br = b_ih[0:E] + b_hh[0:E]
  bz = b_ih[E:2 * E] + b_hh[E:2 * E]
  bn = b_ih[2 * E:3 * E]
  b96 = jnp.concatenate([br, bz, br, bz, bn, bn])[:, None]
  bhn = b_hh[2 * E:3 * E][:, None]
  return W3, b96, bhn


def kernel(users, items, user_memory_init, item_memory_init,
           W_ih, W_hh, b_ih, b_hh):
  B, S = users.shape
  NU = user_memory_init.shape[1]
  NI = item_memory_init.shape[1]
  users = users.astype(jnp.int32)
  items = items.astype(jnp.int32)
  total = B * S

  um_t2d = user_memory_init.transpose(1, 2, 0).reshape(NU * E, B)
  im_t2d = item_memory_init.transpose(1, 2, 0).reshape(NI * E, B)

  gather_k, nw, n_chunks = _make_sc_gather(total, B * NU, B * NI)
  row_off = (jnp.arange(B, dtype=jnp.int32))[:, None]
  uidx = (users + row_off * NU).reshape(nw, n_chunks, CHUNK)
  iidx = (items + row_off * NI).reshape(nw, n_chunks, CHUNK)
  um_p = _pack_table(um_t2d)
  u0p = gather_k(um_p, uidx)
  im_p = _pack_table(im_t2d)
  i0p = gather_k(im_p, iidx)

  W3, b96, bhn = _pack_weights(W_ih, b_ih, b_hh)

  out_u_t, out_i_t = pl.pallas_call(
      _tc_body,
      out_shape=(jax.ShapeDtypeStruct((S * E, B), jnp.float32),
                 jax.ShapeDtypeStruct((S * E, B), jnp.float32)),
  )(u0p.reshape(B, S * E), i0p.reshape(B, S * E), users.T, items.T,
    W3, b96, bhn)

  out_u = out_u_t.reshape(S, E, B).transpose(2, 0, 1)
  out_i = out_i_t.reshape(S, E, B).transpose(2, 0, 1)
  return out_u, out_i
